# Initial kernel scaffold; baseline (speedup 1.0000x reference)
import jax
import jax.numpy as jnp
from jax.experimental import pallas as pl


def kernel(x, W, pre_bias, latent_bias):
    raise NotImplementedError("write your pallas kernel here")


# speedup vs baseline: 11.0198x; 11.0198x over previous
import jax
import jax.numpy as jnp
from jax.experimental import pallas as pl

N = 2048
D = 2048
H = 16384
K = 64

_BH_ENC = 512
_BN_SEL = 128
_BN_DEC = 1024
_BH_DEC = 1024


def _encode_kernel(x_ref, w_ref, pb_ref, lb_ref, out_ref):
    xc = x_ref[...] - pb_ref[...]
    out_ref[...] = jax.lax.dot_general(
        xc, w_ref[...], (((1,), (1,)), ((), ())),
        preferred_element_type=jnp.float32) + lb_ref[...]


def _select_kernel(pa_ref, feat_ref):
    pa = pa_ref[...]
    b = jax.lax.bitcast_convert_type(pa, jnp.int32)
    keys = jnp.where(b >= 0, b, b ^ jnp.int32(0x7FFFFFFF))
    lo = jnp.min(keys, axis=1, keepdims=True)
    hi = jnp.max(keys, axis=1, keepdims=True) + 1

    def body(_, carry):
        lo, hi = carry
        mid = (lo >> 1) + (hi >> 1) + (lo & hi & 1)
        cnt = jnp.sum((keys >= mid).astype(jnp.int32), axis=1, keepdims=True)
        ok = cnt >= K
        return jnp.where(ok, mid, lo), jnp.where(ok, hi, mid)

    lo, hi = jax.lax.fori_loop(0, 32, body, (lo, hi))
    mask = keys >= lo
    feat_ref[...] = jnp.where(mask, jnp.maximum(pa, 0.0), 0.0)


def _decode_kernel(feat_ref, w_ref, pb_ref, out_ref):
    j = pl.program_id(1)
    acc = jax.lax.dot_general(
        feat_ref[...], w_ref[...], (((1,), (0,)), ((), ())),
        preferred_element_type=jnp.float32)

    @pl.when(j == 0)
    def _():
        out_ref[...] = acc + pb_ref[...]

    @pl.when(j != 0)
    def _():
        out_ref[...] += acc


def kernel(x, W, pre_bias, latent_bias):
    pb = pre_bias.reshape(1, D)
    lb = latent_bias.reshape(1, H)

    pre_act = pl.pallas_call(
        _encode_kernel,
        grid=(H // _BH_ENC,),
        in_specs=[
            pl.BlockSpec((N, D), lambda j: (0, 0)),
            pl.BlockSpec((_BH_ENC, D), lambda j: (j, 0)),
            pl.BlockSpec((1, D), lambda j: (0, 0)),
            pl.BlockSpec((1, _BH_ENC), lambda j: (0, j)),
        ],
        out_specs=pl.BlockSpec((N, _BH_ENC), lambda j: (0, j)),
        out_shape=jax.ShapeDtypeStruct((N, H), jnp.float32),
    )(x, W, pb, lb)

    features = pl.pallas_call(
        _select_kernel,
        grid=(N // _BN_SEL,),
        in_specs=[pl.BlockSpec((_BN_SEL, H), lambda i: (i, 0))],
        out_specs=pl.BlockSpec((_BN_SEL, H), lambda i: (i, 0)),
        out_shape=jax.ShapeDtypeStruct((N, H), jnp.float32),
    )(pre_act)

    x_hat = pl.pallas_call(
        _decode_kernel,
        grid=(N // _BN_DEC, H // _BH_DEC),
        in_specs=[
            pl.BlockSpec((_BN_DEC, _BH_DEC), lambda i, j: (i, j)),
            pl.BlockSpec((_BH_DEC, D), lambda i, j: (j, 0)),
            pl.BlockSpec((1, D), lambda i, j: (0, 0)),
        ],
        out_specs=pl.BlockSpec((_BN_DEC, D), lambda i, j: (i, 0)),
        out_shape=jax.ShapeDtypeStruct((N, D), jnp.float32),
    )(features, W, pb)

    return (x_hat, features)
